# baseline (speedup 1.0000x reference)
import jax
import jax.numpy as jnp
from jax.experimental import pallas as pl

_THRESHOLD = 0.5

_BLOCK_ROWS = 2048


def _copy_block(x_ref, o_ref):
    o_ref[...] = x_ref[...]


def kernel(x, heatmap0):
    del heatmap0
    b, c, h, w = x.shape
    rows = b * c * h
    x2 = x.reshape(rows, w)
    grid = rows // _BLOCK_ROWS
    out = pl.pallas_call(
        _copy_block,
        grid=(grid,),
        in_specs=[pl.BlockSpec((_BLOCK_ROWS, w), lambda i: (i, 0))],
        out_specs=pl.BlockSpec((_BLOCK_ROWS, w), lambda i: (i, 0)),
        out_shape=jax.ShapeDtypeStruct((rows, w), x.dtype),
    )(x2)
    return (out.reshape(b, c, h, w), jnp.float32(_THRESHOLD))


# speedup vs baseline: 1.0362x; 1.0362x over previous
import jax
import jax.numpy as jnp
from jax.experimental import pallas as pl

_THRESHOLD = 0.5

_BLOCK_ROWS = 8192


def _copy_block(x_ref, o_ref):
    o_ref[...] = x_ref[...]


def kernel(x, heatmap0):
    del heatmap0
    b, c, h, w = x.shape
    rows = b * c * h
    x2 = x.reshape(rows, w)
    grid = rows // _BLOCK_ROWS
    out = pl.pallas_call(
        _copy_block,
        grid=(grid,),
        in_specs=[pl.BlockSpec((_BLOCK_ROWS, w), lambda i: (i, 0))],
        out_specs=pl.BlockSpec((_BLOCK_ROWS, w), lambda i: (i, 0)),
        out_shape=jax.ShapeDtypeStruct((rows, w), x.dtype),
    )(x2)
    return (out.reshape(b, c, h, w), jnp.float32(_THRESHOLD))
